# baseline (speedup 1.0000x reference)
import functools

import jax
import jax.numpy as jnp
from jax import lax
from jax.experimental import pallas as pl
from jax.experimental.pallas import tpu as pltpu
from jax.experimental.pallas import tpu_sc as plsc

_N_CODES = 100000
_D = 128
_B = 16384

_info = plsc.get_sparse_core_info()
_NC = _info.num_cores
_NS = _info.num_subcores
_NW = _NC * _NS
_B_PER_W = _B // _NW
_CHUNK = 64
_N_CHUNKS = _B_PER_W // _CHUNK


def _make_kernel():
    mesh = plsc.VectorSubcoreMesh(core_axis_name="c", subcore_axis_name="s")

    @functools.partial(
        pl.kernel,
        mesh=mesh,
        out_type=(
            jax.ShapeDtypeStruct((_B, _D), jnp.float32),
            jax.ShapeDtypeStruct((_B, _D), jnp.float32),
        ),
        scratch_types=(
            [pltpu.VMEM((_N_CHUNKS, _CHUNK), jnp.int32)]
            + [pltpu.VMEM((_CHUNK, _D), jnp.float32)] * 14
            + [pltpu.SemaphoreType.DMA] * 28
        ),
    )
    def k(ids_hbm, shape_hbm, tex_hbm, zs_hbm, zt_hbm, idx_v, *scr):
        wid = lax.axis_index("s") * _NC + lax.axis_index("c")
        base = wid * _B_PER_W
        bufs = scr[:14]
        gsem = scr[14:28]
        wsem = scr[28:42]
        pltpu.sync_copy(ids_hbm.at[wid], idx_v)
        tasks = []
        for c in range(_N_CHUNKS):
            tasks.append((shape_hbm, zs_hbm, c))
            tasks.append((tex_hbm, zt_hbm, c))
        nt = len(tasks)
        nbuf = len(bufs)
        gcps = [None] * nt
        wcps = [None] * nt
        for i in range(nbuf):
            tbl, _, c = tasks[i]
            gcps[i] = pltpu.async_copy(tbl.at[idx_v.at[c]], bufs[i], gsem[i])
        for i in range(nt):
            _, out, c = tasks[i]
            b = i % nbuf
            if i >= nbuf:
                tbl, _, c_i = tasks[i]
                wcps[b].wait()
                gcps[i] = pltpu.async_copy(
                    tbl.at[idx_v.at[c_i]], bufs[b], gsem[b])
            gcps[i].wait()
            wcps[i] = pltpu.async_copy(
                bufs[b], out.at[pl.ds(base + c * _CHUNK, _CHUNK)], wsem[b])
        for i in range(nt - nbuf, nt):
            wcps[i].wait()

    return k


_gather2 = _make_kernel()


def kernel(object_ids, shape_table, texture_table):
    ids = object_ids.astype(jnp.int32).reshape(_NW, _N_CHUNKS, _CHUNK)
    z_s, z_t = _gather2(ids, shape_table, texture_table)
    return (z_s, z_t)


# speedup vs baseline: 1.0283x; 1.0283x over previous
import functools

import jax
import jax.numpy as jnp
from jax import lax
from jax.experimental import pallas as pl
from jax.experimental.pallas import tpu as pltpu
from jax.experimental.pallas import tpu_sc as plsc

_N_CODES = 100000
_D = 128
_B = 16384

_info = plsc.get_sparse_core_info()
_NC = _info.num_cores
_NS = _info.num_subcores
_NW = _NC * _NS
_B_PER_W = _B // _NW
_CHUNK = 128
_N_CHUNKS = _B_PER_W // _CHUNK
_NBUF = 7


def _make_kernel():
    mesh = plsc.VectorSubcoreMesh(core_axis_name="c", subcore_axis_name="s")

    @functools.partial(
        pl.kernel,
        mesh=mesh,
        out_type=(
            jax.ShapeDtypeStruct((_B, _D), jnp.float32),
            jax.ShapeDtypeStruct((_B, _D), jnp.float32),
        ),
        scratch_types=(
            [pltpu.VMEM((_N_CHUNKS, _CHUNK), jnp.int32)]
            + [pltpu.VMEM((_CHUNK, _D), jnp.float32)] * _NBUF
            + [pltpu.SemaphoreType.DMA] * (_N_CHUNKS + 2 * _NBUF)
        ),
    )
    def k(ids_hbm, shape_hbm, tex_hbm, zs_hbm, zt_hbm, idx_v, *scr):
        wid = lax.axis_index("c") * _NS + lax.axis_index("s")
        base = wid * _B_PER_W
        bufs = scr[:_NBUF]
        isem = scr[_NBUF:_NBUF + _N_CHUNKS]
        gsem = scr[_NBUF + _N_CHUNKS:2 * _NBUF + _N_CHUNKS]
        wsem = scr[2 * _NBUF + _N_CHUNKS:3 * _NBUF + _N_CHUNKS]
        icps = [
            pltpu.async_copy(ids_hbm.at[wid, c], idx_v.at[c], isem[c])
            for c in range(_N_CHUNKS)
        ]
        tasks = []
        for c in range(_N_CHUNKS):
            tasks.append((shape_hbm, zs_hbm, c))
            tasks.append((tex_hbm, zt_hbm, c))
        nt = len(tasks)
        gcps = [None] * nt
        wcps = [None] * nt
        for i in range(_NBUF):
            tbl, _, c = tasks[i]
            if i % 2 == 0:
                icps[c].wait()
            gcps[i] = pltpu.async_copy(tbl.at[idx_v.at[c]], bufs[i], gsem[i])
        for i in range(nt):
            _, out, c = tasks[i]
            b = i % _NBUF
            if i >= _NBUF:
                tbl, _, c_i = tasks[i]
                wcps[b].wait()
                gcps[i] = pltpu.async_copy(
                    tbl.at[idx_v.at[c_i]], bufs[b], gsem[b])
            gcps[i].wait()
            wcps[i] = pltpu.async_copy(
                bufs[b], out.at[pl.ds(base + c * _CHUNK, _CHUNK)], wsem[b])
        for i in range(nt - _NBUF, nt):
            wcps[i].wait()

    return k


_gather2 = _make_kernel()


def kernel(object_ids, shape_table, texture_table):
    ids = object_ids.astype(jnp.int32).reshape(_NW, _N_CHUNKS, _CHUNK)
    z_s, z_t = _gather2(ids, shape_table, texture_table)
    return (z_s, z_t)


# speedup vs baseline: 1.0451x; 1.0164x over previous
import functools

import jax
import jax.numpy as jnp
from jax import lax
from jax.experimental import pallas as pl
from jax.experimental.pallas import tpu as pltpu
from jax.experimental.pallas import tpu_sc as plsc

_N_CODES = 100000
_D = 128
_B = 16384

_info = plsc.get_sparse_core_info()
_NC = _info.num_cores
_NS = _info.num_subcores
_NW = _NC * _NS
_B_PER_W = _B // _NW
_CHUNK = 128
_N_CHUNKS = _B_PER_W // _CHUNK


def _make_kernel():
    mesh = plsc.VectorSubcoreMesh(core_axis_name="c", subcore_axis_name="s")

    @functools.partial(
        pl.kernel,
        mesh=mesh,
        out_type=(
            jax.ShapeDtypeStruct((_B, _D), jnp.float32),
            jax.ShapeDtypeStruct((_B, _D), jnp.float32),
        ),
        scratch_types=(
            [pltpu.VMEM((_N_CHUNKS, _CHUNK), jnp.int32)]
            + [pltpu.VMEM((_CHUNK, _D), jnp.float32)] * 7
            + [pltpu.SemaphoreType.DMA] * 14
        ),
    )
    def k(ids_hbm, shape_hbm, tex_hbm, zs_hbm, zt_hbm, idx_v, *scr):
        wid = lax.axis_index("s") * _NC + lax.axis_index("c")
        base = wid * _B_PER_W
        bufs = scr[:7]
        gsem = scr[7:14]
        wsem = scr[14:21]
        pltpu.sync_copy(ids_hbm.at[wid], idx_v)
        tasks = []
        for c in range(_N_CHUNKS):
            tasks.append((shape_hbm, zs_hbm, c))
            tasks.append((tex_hbm, zt_hbm, c))
        nt = len(tasks)
        nbuf = len(bufs)
        gcps = [None] * nt
        wcps = [None] * nt
        for i in range(nbuf):
            tbl, _, c = tasks[i]
            gcps[i] = pltpu.async_copy(tbl.at[idx_v.at[c]], bufs[i], gsem[i])
        for i in range(nt):
            _, out, c = tasks[i]
            b = i % nbuf
            if i >= nbuf:
                tbl, _, c_i = tasks[i]
                wcps[b].wait()
                gcps[i] = pltpu.async_copy(
                    tbl.at[idx_v.at[c_i]], bufs[b], gsem[b])
            gcps[i].wait()
            wcps[i] = pltpu.async_copy(
                bufs[b], out.at[pl.ds(base + c * _CHUNK, _CHUNK)], wsem[b])
        for i in range(nt - nbuf, nt):
            wcps[i].wait()

    return k


_gather2 = _make_kernel()


def kernel(object_ids, shape_table, texture_table):
    ids = object_ids.astype(jnp.int32).reshape(_NW, _N_CHUNKS, _CHUNK)
    z_s, z_t = _gather2(ids, shape_table, texture_table)
    return (z_s, z_t)
